# Initial kernel scaffold; baseline (speedup 1.0000x reference)
import jax
import jax.numpy as jnp
from jax.experimental import pallas as pl


def kernel(x, idx_token, agg_weight):
    raise NotImplementedError("write your pallas kernel here")


# speedup vs baseline: 2.2918x; 2.2918x over previous
import functools
import math

import jax
import jax.numpy as jnp
from jax.experimental import pallas as pl
from jax.experimental.pallas import tpu as pltpu


def _ctm_fused_kernel(x_ref, xm_ref, idx_ref, nw_ref, cid_ref, *, k, S, C):
    N = x_ref.shape[1]
    xb = x_ref[0]
    sq = jnp.sum(xb * xb, axis=-1, keepdims=True)
    gram = jax.lax.dot_general(
        xb, xb, (((1,), (1,)), ((), ())),
        preferred_element_type=jnp.float32)
    d2 = jnp.maximum(sq + sq.T - 2.0 * gram, 0.0)

    iota_col = jax.lax.broadcasted_iota(jnp.int32, (N, N), 1)
    iota_row = jax.lax.broadcasted_iota(jnp.int32, (N, N), 0)
    n_i = jnp.int32(N)

    cur = d2
    acc = jnp.zeros((N, 1), jnp.float32)
    for _ in range(k):
        m = jnp.min(cur, axis=-1, keepdims=True)
        acc = acc + m
        first = jnp.min(jnp.where(cur == m, iota_col, n_i),
                        axis=-1, keepdims=True)
        cur = jnp.where(iota_col == first, jnp.inf, cur)

    density = jnp.exp(-acc * (1.0 / (k * C)))
    density_row = density.T
    d2max = jnp.max(d2)
    val = jnp.where(density_row > density, d2, d2max)
    dist2 = jnp.min(val, axis=-1, keepdims=True)
    dist = jnp.sqrt(dist2) * (1.0 / math.sqrt(C))
    score = dist * density
    score_row = score.T

    beats = (score > score_row) | ((score == score_row) & (iota_row < iota_col))
    rank_row = jnp.sum(beats.astype(jnp.int32), axis=0, keepdims=True)
    beats_t = (score_row > score) | ((score_row == score) & (iota_col < iota_row))
    rank = jnp.sum(beats_t.astype(jnp.int32), axis=-1, keepdims=True)

    iota_s_cols = jax.lax.broadcasted_iota(jnp.int32, (N, S), 1)
    iota_n_rows = jax.lax.broadcasted_iota(jnp.int32, (N, S), 0)
    sel = rank == iota_s_cols
    index_down = jnp.min(jnp.where(sel, iota_n_rows, n_i),
                         axis=0, keepdims=True)
    cid_ref[0] = index_down

    elig = rank < S
    mcol = jnp.min(jnp.where(elig, d2, jnp.inf), axis=0, keepdims=True)
    assign = jnp.min(jnp.where(elig & (d2 == mcol), rank, jnp.int32(S)),
                     axis=0, keepdims=True)
    idx_final = jnp.where(rank_row < S, rank_row, assign)
    idx_ref[0] = idx_final

    iota_s_rows = jax.lax.broadcasted_iota(jnp.int32, (S, N), 0)
    onehot = (idx_final == iota_s_rows).astype(jnp.float32)
    counts = jnp.sum(onehot, axis=-1, keepdims=True)
    inv = 1.0 / (counts + 1e-6)
    xsum = jax.lax.dot_general(
        onehot, xb, (((1,), (0,)), ((), ())),
        preferred_element_type=jnp.float32)
    xm_ref[0] = xsum * inv
    nw_ref[0] = jnp.sum(onehot * inv, axis=0, keepdims=True)


def kernel(x, idx_token, agg_weight):
    B, N, C = x.shape
    S = max(math.ceil(N * 0.25), 1)
    k = 5 if 5 <= S else min(3, max(S // 2, 1))

    itemsize = jnp.dtype(x.dtype).itemsize
    cost = pl.CostEstimate(
        flops=B * (2 * N * N * C + 2 * S * N * C + (9 + 3 * k) * N * N),
        transcendentals=2 * B * N,
        bytes_accessed=B * (N * C * itemsize + S * C * 4 + 3 * N * 4 + S * 4),
    )
    xm, idxc, nw, cid = pl.pallas_call(
        functools.partial(_ctm_fused_kernel, k=k, S=S, C=C),
        out_shape=(
            jax.ShapeDtypeStruct((B, S, C), jnp.float32),
            jax.ShapeDtypeStruct((B, 1, N), jnp.int32),
            jax.ShapeDtypeStruct((B, 1, N), jnp.float32),
            jax.ShapeDtypeStruct((B, 1, S), jnp.int32),
        ),
        grid=(B,),
        in_specs=[pl.BlockSpec((1, N, C), lambda i: (i, 0, 0))],
        out_specs=(
            pl.BlockSpec((1, S, C), lambda i: (i, 0, 0)),
            pl.BlockSpec((1, 1, N), lambda i: (i, 0, 0)),
            pl.BlockSpec((1, 1, N), lambda i: (i, 0, 0)),
            pl.BlockSpec((1, 1, S), lambda i: (i, 0, 0)),
        ),
        compiler_params=pltpu.CompilerParams(
            dimension_semantics=("parallel",),
            vmem_limit_bytes=56 * 1024 * 1024),
        cost_estimate=cost,
    )(x)

    idx_cluster = idxc[:, 0, :]
    normw = nw[:, 0, :]
    idx_token_new = jnp.take_along_axis(idx_cluster, idx_token, axis=1)
    agg_weight_new = agg_weight * jnp.take_along_axis(
        normw, idx_token, axis=1)[..., None]
    return xm, idx_token_new, agg_weight_new, cid[:, 0, :]
